# baseline (speedup 1.0000x reference)
import functools

import jax
import jax.numpy as jnp
from jax.experimental import pallas as pl
from jax.experimental.pallas import tpu as pltpu

_BLOCK_G = 200


def _body(num_blocks,
          at_ref, gxb_ref, cxt_ref,
          wcs_ref, wcn_ref, bc_ref,
          wgs_ref, wgn_ref, bg_ref,
          gene_out_ref, cot_ref,
          part_a_ref, part_b_ref):
    i = pl.program_id(0)

    at = at_ref[...]
    gxb = gxb_ref[...]

    gn_blk = jax.lax.dot_general(
        at, cxt_ref[...], (((1,), (1,)), ((), ())),
        precision=jax.lax.Precision.DEFAULT,
        preferred_element_type=jnp.float32)
    gene_out_ref[...] = jnp.maximum(
        jnp.dot(gxb, wgs_ref[...], preferred_element_type=jnp.float32)
        + jnp.dot(gn_blk, wgn_ref[...], preferred_element_type=jnp.float32)
        + bg_ref[...],
        0.0)

    part = jax.lax.dot_general(
        gxb, at,
        (((0,), (0,)), ((), ())),
        precision=jax.lax.Precision.DEFAULT,
        preferred_element_type=jnp.float32)

    @pl.when(i % 2 == 0)
    def _store_a():
        part_a_ref[...] = part

    @pl.when(i % 2 == 1)
    def _store_b():
        part_b_ref[...] = part

    @pl.when(i == 1)
    def _init_acc():
        cot_ref[...] = part_a_ref[...]

    @pl.when((i > 1) & (i % 2 == 1))
    def _fold_a():
        cot_ref[...] += part_a_ref[...]

    @pl.when((i > 1) & (i % 2 == 0))
    def _fold_b():
        cot_ref[...] += part_b_ref[...]

    @pl.when(i == num_blocks - 1)
    def _finish():
        if num_blocks % 2 == 0:
            last_part = part_b_ref[...]
        else:
            last_part = part_a_ref[...]
        cn_t = cot_ref[...] + last_part
        cot_ref[...] = jnp.maximum(
            jnp.dot(wcs_ref[...], cxt_ref[...],
                    preferred_element_type=jnp.float32)
            + jnp.dot(wcn_ref[...], cn_t,
                      preferred_element_type=jnp.float32)
            + bc_ref[...],
            0.0)


def kernel(cell_x, gene_x, adjacency,
           W_cell_self, b_cell_self, W_cell_neigh, b_cell_neigh,
           W_gene_self, b_gene_self, W_gene_neigh, b_gene_neigh):
    N_c, d = cell_x.shape
    N_g = gene_x.shape[0]

    num_blocks = N_g // _BLOCK_G
    assert num_blocks * _BLOCK_G == N_g and num_blocks >= 3

    at = adjacency.T
    cxt = cell_x.T
    wgs = W_gene_self.T
    wgn = W_gene_neigh.T
    bc = (b_cell_self + b_cell_neigh).reshape(d, 1)
    bg = (b_gene_self + b_gene_neigh).reshape(1, d)

    full = lambda shape: pl.BlockSpec(shape, lambda i: (0, 0))

    gene_out, cot = pl.pallas_call(
        functools.partial(_body, num_blocks),
        grid=(num_blocks,),
        in_specs=[
            pl.BlockSpec((_BLOCK_G, N_c), lambda i: (i, 0)),
            pl.BlockSpec((_BLOCK_G, d), lambda i: (i, 0)),
            full((d, N_c)),
            full((d, d)), full((d, d)), full((d, 1)),
            full((d, d)), full((d, d)), full((1, d)),
        ],
        out_specs=[
            pl.BlockSpec((_BLOCK_G, d), lambda i: (i, 0)),
            full((d, N_c)),
        ],
        out_shape=[
            jax.ShapeDtypeStruct((N_g, d), jnp.float32),
            jax.ShapeDtypeStruct((d, N_c), jnp.float32),
        ],
        scratch_shapes=[
            pltpu.VMEM((d, N_c), jnp.float32),
            pltpu.VMEM((d, N_c), jnp.float32),
        ],
        compiler_params=pltpu.CompilerParams(
            dimension_semantics=("arbitrary",),
        ),
    )(at, gene_x, cxt,
      W_cell_self, W_cell_neigh, bc, wgs, wgn, bg)

    return (cot.T, gene_out)


# speedup vs baseline: 1.0442x; 1.0442x over previous
import functools

import jax
import jax.numpy as jnp
from jax.experimental import pallas as pl
from jax.experimental.pallas import tpu as pltpu

_BLOCK_G = 200


def _body(num_blocks,
          at_ref, gxb_ref, cxt_ref,
          wcs_ref, wcn_ref, bc_ref,
          wgs_ref, wgn_ref, bg_ref,
          gene_out_ref, cot_ref,
          acc_ref):
    i = pl.program_id(0)

    at = at_ref[...]
    gxb = gxb_ref[...]

    gn_blk = jax.lax.dot_general(
        at, cxt_ref[...], (((1,), (1,)), ((), ())),
        precision=jax.lax.Precision.DEFAULT,
        preferred_element_type=jnp.float32)
    gene_out_ref[...] = jnp.maximum(
        jnp.dot(gxb, wgs_ref[...], preferred_element_type=jnp.float32)
        + jnp.dot(gn_blk, wgn_ref[...], preferred_element_type=jnp.float32)
        + bg_ref[...],
        0.0)

    part = jax.lax.dot_general(
        gxb, at,
        (((0,), (0,)), ((), ())),
        precision=jax.lax.Precision.DEFAULT,
        preferred_element_type=jnp.float32)

    @pl.when(i == 0)
    def _init():
        acc_ref[...] = part

    @pl.when(i > 0)
    def _accum():
        acc_ref[...] += part

    @pl.when(i == num_blocks - 1)
    def _finish():
        cot_ref[...] = jnp.maximum(
            jnp.dot(wcs_ref[...], cxt_ref[...],
                    preferred_element_type=jnp.float32)
            + jnp.dot(wcn_ref[...], acc_ref[...],
                      preferred_element_type=jnp.float32)
            + bc_ref[...],
            0.0)


def kernel(cell_x, gene_x, adjacency,
           W_cell_self, b_cell_self, W_cell_neigh, b_cell_neigh,
           W_gene_self, b_gene_self, W_gene_neigh, b_gene_neigh):
    N_c, d = cell_x.shape
    N_g = gene_x.shape[0]

    num_blocks = N_g // _BLOCK_G
    assert num_blocks * _BLOCK_G == N_g

    at = adjacency.T
    cxt = cell_x.T
    wgs = W_gene_self.T
    wgn = W_gene_neigh.T
    bc = (b_cell_self + b_cell_neigh).reshape(d, 1)
    bg = (b_gene_self + b_gene_neigh).reshape(1, d)

    full = lambda shape: pl.BlockSpec(shape, lambda i: (0, 0))

    gene_out, cot = pl.pallas_call(
        functools.partial(_body, num_blocks),
        grid=(num_blocks,),
        in_specs=[
            pl.BlockSpec((_BLOCK_G, N_c), lambda i: (i, 0)),
            pl.BlockSpec((_BLOCK_G, d), lambda i: (i, 0)),
            full((d, N_c)),
            full((d, d)), full((d, d)), full((d, 1)),
            full((d, d)), full((d, d)), full((1, d)),
        ],
        out_specs=[
            pl.BlockSpec((_BLOCK_G, d), lambda i: (i, 0)),
            full((d, N_c)),
        ],
        out_shape=[
            jax.ShapeDtypeStruct((N_g, d), jnp.float32),
            jax.ShapeDtypeStruct((d, N_c), jnp.float32),
        ],
        scratch_shapes=[
            pltpu.VMEM((d, N_c), jnp.float32),
        ],
        compiler_params=pltpu.CompilerParams(
            dimension_semantics=("arbitrary",),
        ),
    )(at, gene_x, cxt,
      W_cell_self, W_cell_neigh, bc, wgs, wgn, bg)

    return (cot.T, gene_out)
